# baseline (speedup 1.0000x reference)
import functools
import math

import jax
import jax.numpy as jnp
import numpy as np
from jax import lax
from jax.experimental import pallas as pl
from jax.experimental.pallas import tpu as pltpu
from jax.experimental.pallas import tpu_sc as plsc

NLEV = 16
TABLE_SIZE = 262144
TMASK = TABLE_SIZE - 1
NPTS = 262144
NOUT = 2 * NLEV

P1 = np.uint32(2654435761).view(np.int32).item()
P2 = np.uint32(805459861).view(np.int32).item()


def _level_res():
    minres = np.array([16.0, 16.0, 16.0], dtype=np.float64)
    maxres = np.array([512.0, 512.0, 512.0], dtype=np.float64)
    b = np.exp((np.log(maxres) - np.log(minres)) / (NLEV - 1))
    return [int(np.floor(minres * b**l).astype(np.int64)[0]) for l in range(NLEV)]


RES = _level_res()

NW = 32
PPW = NPTS // NW
C = 1024
NCHUNK = PPW // C
NG = C // 16

_i32 = jnp.int32
_f32 = jnp.float32


def _iota16():
    return lax.iota(_i32, 16)


def _round_half_even(u):
    t = u + 0.5
    r = t.astype(_i32)
    tie = (r.astype(_f32) == t) & ((r & 1) == 1)
    return jnp.where(tie, r - 1, r)


def _body(coords_hbm, grid_hbm, tables_hbm, out_hbm,
          grid_v, cx_v, cy_v, cz_v, idx_v, w_v, rows_v, out_v,
          sem0):
    wid = lax.axis_index("s") * 2 + lax.axis_index("c")

    pltpu.sync_copy(grid_hbm, grid_v)

    def chunk_body(ch, _):
        base = wid * PPW + ch * C
        pltpu.sync_copy(coords_hbm.at[pl.ds(0 * NPTS + base, C)], cx_v)
        pltpu.sync_copy(coords_hbm.at[pl.ds(1 * NPTS + base, C)], cy_v)
        pltpu.sync_copy(coords_hbm.at[pl.ds(2 * NPTS + base, C)], cz_v)

        def g0(g, _):
            p = g * 16
            x = cx_v[pl.ds(p, 16)]
            y = cy_v[pl.ds(p, 16)]
            z = cz_v[pl.ds(p, 16)]
            r1f = float(RES[0] - 1)
            ix = _round_half_even((x + 1.0) * 0.5 * r1f)
            iy = _round_half_even((y + 1.0) * 0.5 * r1f)
            iz = _round_half_even((z + 1.0) * 0.5 * r1f)
            gi = (ix * (RES[0] * RES[0]) + iy * RES[0] + iz) * 2
            f0 = plsc.load_gather(grid_v, [gi])
            f1 = plsc.load_gather(grid_v, [gi + 1])
            oi = (p + _iota16()) * NOUT
            plsc.store_scatter(out_v, [oi], f0)
            plsc.store_scatter(out_v, [oi + 1], f1)
            return 0

        lax.fori_loop(0, NG, g0, 0)

        for l in range(1, NLEV):
            res = RES[l]
            rm1f = float(res - 1)
            rm1 = res - 1

            loff = (l - 1) * TABLE_SIZE

            def ga(g, _, rm1f=rm1f, rm1=rm1, loff=loff):
                p = g * 16
                x = cx_v[pl.ds(p, 16)]
                y = cy_v[pl.ds(p, 16)]
                z = cz_v[pl.ds(p, 16)]
                ux = (x + 1.0) * 0.5 * rm1f
                uy = (y + 1.0) * 0.5 * rm1f
                uz = (z + 1.0) * 0.5 * rm1f
                fx = ux.astype(_i32)
                fy = uy.astype(_i32)
                fz = uz.astype(_i32)
                wx = ux - fx.astype(_f32)
                wy = uy - fy.astype(_f32)
                wz = uz - fz.astype(_f32)
                x1 = jnp.minimum(fx + 1, rm1)
                y1 = jnp.minimum(fy + 1, rm1)
                z1 = jnp.minimum(fz + 1, rm1)
                hx = (fx, x1)
                hy = (fy * P1, y1 * P1)
                hz = (fz * P2, z1 * P2)
                ox = (1.0 - wx, wx)
                oy = (1.0 - wy, wy)
                oz = (1.0 - wz, wz)
                for ci, (dx, dy, dz) in enumerate(
                        [(a, b, c) for a in (0, 1) for b in (0, 1) for c in (0, 1)]):
                    h = (hx[dx] ^ hy[dy] ^ hz[dz]) & TMASK
                    wc = (ox[dx] * oy[dy]) * oz[dz]
                    idx_v[pl.ds(ci * C + p, 16)] = h | loff
                    w_v[pl.ds(ci * C + p, 16)] = wc
                return 0

            lax.fori_loop(0, NG, ga, 0)

            d0 = pltpu.async_copy(tables_hbm.at[idx_v], rows_v, sem0)
            d0.wait()

            def gc(g, _, l=l):
                p = g * 16
                iota = _iota16()
                acc0 = jnp.zeros((16,), _f32)
                acc1 = jnp.zeros((16,), _f32)
                zeros = jnp.zeros((16,), _i32)
                for ci in range(8):
                    w = w_v[pl.ds(ci * C + p, 16)]
                    ridx = ci * C + p + iota
                    acc0 = acc0 + w * plsc.load_gather(rows_v, [ridx, zeros])
                    acc1 = acc1 + w * plsc.load_gather(rows_v, [ridx, zeros + 1])
                oi = (p + iota) * NOUT + (2 * l)
                plsc.store_scatter(out_v, [oi], acc0)
                plsc.store_scatter(out_v, [oi + 1], acc1)
                return 0

            lax.fori_loop(0, NG, gc, 0)

        pltpu.sync_copy(out_v, out_hbm.at[pl.ds(base * NOUT, C * NOUT)])
        return 0

    lax.fori_loop(0, NCHUNK, chunk_body, 0)


@functools.partial(
    pl.kernel,
    out_type=jax.ShapeDtypeStruct((NPTS * NOUT,), _f32),
    mesh=plsc.VectorSubcoreMesh(core_axis_name="c", subcore_axis_name="s"),
    compiler_params=pltpu.CompilerParams(
        needs_layout_passes=False, use_tc_tiling_on_sc=False),
    scratch_types=[
        pltpu.VMEM((RES[0] ** 3 * 2,), _f32),
        pltpu.VMEM((C,), _f32),
        pltpu.VMEM((C,), _f32),
        pltpu.VMEM((C,), _f32),
        pltpu.VMEM((8 * C,), _i32),
        pltpu.VMEM((8 * C,), _f32),
        pltpu.VMEM((8 * C, 2), _f32),
        pltpu.VMEM((C * NOUT,), _f32),
        pltpu.SemaphoreType.DMA,
    ],
)
def _sc_encode(coords_hbm, grid_hbm, tables_hbm, out_hbm, *scratch):
    _body(coords_hbm, grid_hbm, tables_hbm, out_hbm, *scratch)


def kernel(coords, grid0, tables):
    coords_t = coords.T.reshape(-1)
    grid_r = grid0.transpose(1, 2, 3, 0).reshape(-1)
    tables_p = tables.transpose(0, 2, 1).reshape(NLEV - 1, TABLE_SIZE, 2)
    tables_p = tables_p.reshape((NLEV - 1) * TABLE_SIZE, 2)
    out = _sc_encode(coords_t, grid_r, tables_p)
    return out.reshape(NPTS, NOUT)


# speedup vs baseline: 3.2386x; 3.2386x over previous
import functools
import math

import jax
import jax.numpy as jnp
import numpy as np
from jax import lax
from jax.experimental import pallas as pl
from jax.experimental.pallas import tpu as pltpu
from jax.experimental.pallas import tpu_sc as plsc

NLEV = 16
TABLE_SIZE = 262144
TMASK = TABLE_SIZE - 1
NPTS = 262144
NOUT = 2 * NLEV

P1 = np.uint32(2654435761).view(np.int32).item()
P2 = np.uint32(805459861).view(np.int32).item()


def _level_res():
    minres = np.array([16.0, 16.0, 16.0], dtype=np.float64)
    maxres = np.array([512.0, 512.0, 512.0], dtype=np.float64)
    b = np.exp((np.log(maxres) - np.log(minres)) / (NLEV - 1))
    return [int(np.floor(minres * b**l).astype(np.int64)[0]) for l in range(NLEV)]


RES = _level_res()

NW = 32
PPW = NPTS // NW
C = 512
NCHUNK = PPW // C
NG = C // 16

_i32 = jnp.int32
_f32 = jnp.float32


def _iota16():
    return lax.iota(_i32, 16)


def _round_half_even(u):
    t = u + 0.5
    r = t.astype(_i32)
    tie = (r.astype(_f32) == t) & ((r & 1) == 1)
    return jnp.where(tie, r - 1, r)


def _body(coords_hbm, grid_hbm, tables_hbm, out_hbm,
          grid_v, cx_v, cy_v, cz_v,
          idx_a, idx_b, w_a, w_b, r0_a, r0_b, r1_a, r1_b, out_v,
          sem_a, sem_b, sem_a1, sem_b1):
    wid = lax.axis_index("s") * 2 + lax.axis_index("c")
    bufs = ((idx_a, w_a, r0_a, r1_a, sem_a, sem_a1),
            (idx_b, w_b, r0_b, r1_b, sem_b, sem_b1))

    pltpu.sync_copy(grid_hbm, grid_v)

    def index_gen(l, idx_v, w_v):
        res = RES[l]
        rm1f = float(res - 1)
        rm1 = res - 1

        def ga(g, _):
            p = g * 16
            x = cx_v[pl.ds(p, 16)]
            y = cy_v[pl.ds(p, 16)]
            z = cz_v[pl.ds(p, 16)]
            ux = x * rm1f
            uy = y * rm1f
            uz = z * rm1f
            fx = ux.astype(_i32)
            fy = uy.astype(_i32)
            fz = uz.astype(_i32)
            wx = ux - fx.astype(_f32)
            wy = uy - fy.astype(_f32)
            wz = uz - fz.astype(_f32)
            x1 = jnp.minimum(fx + 1, rm1)
            y1 = jnp.minimum(fy + 1, rm1)
            z1 = jnp.minimum(fz + 1, rm1)
            hx = (fx, x1)
            hy = (fy * P1, y1 * P1)
            hz = (fz * P2, z1 * P2)
            ox = (1.0 - wx, wx)
            oy = (1.0 - wy, wy)
            oz = (1.0 - wz, wz)
            for ci, (dx, dy, dz) in enumerate(
                    [(a, b, c) for a in (0, 1) for b in (0, 1) for c in (0, 1)]):
                h = (hx[dx] ^ hy[dy] ^ hz[dz]) & TMASK
                wc = (ox[dx] * oy[dy]) * oz[dz]
                idx_v[pl.ds(ci * C + p, 16)] = h
                w_v[pl.ds(ci * C + p, 16)] = wc
            return 0

        lax.fori_loop(0, NG, ga, 0)

    def accumulate(l, w_v, r0_v, r1_v):
        def gc(g, _):
            p = g * 16
            iota = _iota16()
            acc0 = jnp.zeros((16,), _f32)
            acc1 = jnp.zeros((16,), _f32)
            for ci in range(8):
                w = w_v[pl.ds(ci * C + p, 16)]
                acc0 = acc0 + w * r0_v[pl.ds(ci * C + p, 16)]
                acc1 = acc1 + w * r1_v[pl.ds(ci * C + p, 16)]
            oi = (p + iota) * NOUT + (2 * l)
            plsc.store_scatter(out_v, [oi], acc0)
            plsc.store_scatter(out_v, [oi + 1], acc1)
            return 0

        lax.fori_loop(0, NG, gc, 0)

    def level0():
        def g0(g, _):
            p = g * 16
            x = cx_v[pl.ds(p, 16)]
            y = cy_v[pl.ds(p, 16)]
            z = cz_v[pl.ds(p, 16)]
            r1f = float(RES[0] - 1)
            ix = _round_half_even(x * r1f)
            iy = _round_half_even(y * r1f)
            iz = _round_half_even(z * r1f)
            gi = (ix * (RES[0] * RES[0]) + iy * RES[0] + iz) * 2
            f0 = plsc.load_gather(grid_v, [gi])
            f1 = plsc.load_gather(grid_v, [gi + 1])
            oi = (p + _iota16()) * NOUT
            plsc.store_scatter(out_v, [oi], f0)
            plsc.store_scatter(out_v, [oi + 1], f1)
            return 0

        lax.fori_loop(0, NG, g0, 0)

    def chunk_body(ch, _):
        base = wid * PPW + ch * C
        pltpu.sync_copy(coords_hbm.at[pl.ds(0 * NPTS + base, C)], cx_v)
        pltpu.sync_copy(coords_hbm.at[pl.ds(1 * NPTS + base, C)], cy_v)
        pltpu.sync_copy(coords_hbm.at[pl.ds(2 * NPTS + base, C)], cz_v)

        def norm(g, _):
            p = g * 16
            for r in (cx_v, cy_v, cz_v):
                r[pl.ds(p, 16)] = (r[pl.ds(p, 16)] + 1.0) * 0.5
            return 0

        lax.fori_loop(0, NG, norm, 0)

        def fire(l, idx_v, r0_v, r1_v, s0, s1):
            o0 = (2 * (l - 1)) * TABLE_SIZE
            o1 = (2 * (l - 1) + 1) * TABLE_SIZE
            d0 = pltpu.async_copy(
                tables_hbm.at[pl.ds(o0, TABLE_SIZE)].at[idx_v], r0_v, s0)
            d1 = pltpu.async_copy(
                tables_hbm.at[pl.ds(o1, TABLE_SIZE)].at[idx_v], r1_v, s1)
            return d0, d1

        idx0, w0, r00, r10, s00, s10 = bufs[0]
        index_gen(1, idx0, w0)
        dmas = fire(1, idx0, r00, r10, s00, s10)
        level0()
        for l in range(1, NLEV):
            _, w_cur, r0_cur, r1_cur, _, _ = bufs[(l - 1) % 2]
            if l < NLEV - 1:
                idx_n, w_n, r0_n, r1_n, s0_n, s1_n = bufs[l % 2]
                index_gen(l + 1, idx_n, w_n)
            dmas[0].wait()
            dmas[1].wait()
            if l < NLEV - 1:
                dmas = fire(l + 1, idx_n, r0_n, r1_n, s0_n, s1_n)
            accumulate(l, w_cur, r0_cur, r1_cur)

        pltpu.sync_copy(out_v, out_hbm.at[pl.ds(base * NOUT, C * NOUT)])
        return 0

    lax.fori_loop(0, NCHUNK, chunk_body, 0)


@functools.partial(
    pl.kernel,
    out_type=jax.ShapeDtypeStruct((NPTS * NOUT,), _f32),
    mesh=plsc.VectorSubcoreMesh(core_axis_name="c", subcore_axis_name="s"),
    compiler_params=pltpu.CompilerParams(needs_layout_passes=False),
    scratch_types=[
        pltpu.VMEM((RES[0] ** 3 * 2,), _f32),
        pltpu.VMEM((C,), _f32),
        pltpu.VMEM((C,), _f32),
        pltpu.VMEM((C,), _f32),
        pltpu.VMEM((8 * C,), _i32),
        pltpu.VMEM((8 * C,), _i32),
        pltpu.VMEM((8 * C,), _f32),
        pltpu.VMEM((8 * C,), _f32),
        pltpu.VMEM((8 * C,), _f32),
        pltpu.VMEM((8 * C,), _f32),
        pltpu.VMEM((8 * C,), _f32),
        pltpu.VMEM((8 * C,), _f32),
        pltpu.VMEM((C * NOUT,), _f32),
        pltpu.SemaphoreType.DMA,
        pltpu.SemaphoreType.DMA,
        pltpu.SemaphoreType.DMA,
        pltpu.SemaphoreType.DMA,
    ],
)
def _sc_encode(coords_hbm, grid_hbm, tables_hbm, out_hbm, *scratch):
    _body(coords_hbm, grid_hbm, tables_hbm, out_hbm, *scratch)


def kernel(coords, grid0, tables):
    coords_t = coords.T.reshape(-1)
    grid_r = grid0.transpose(1, 2, 3, 0).reshape(-1)
    tables_f = tables.reshape(-1)
    out = _sc_encode(coords_t, grid_r, tables_f)
    return out.reshape(NPTS, NOUT)


# speedup vs baseline: 9.9838x; 3.0828x over previous
import functools
import math

import jax
import jax.numpy as jnp
import numpy as np
from jax import lax
from jax.experimental import pallas as pl
from jax.experimental.pallas import tpu as pltpu
from jax.experimental.pallas import tpu_sc as plsc

NLEV = 16
TABLE_SIZE = 262144
TMASK = TABLE_SIZE - 1
NPTS = 262144
NOUT = 2 * NLEV

P1 = np.uint32(2654435761).view(np.int32).item()
P2 = np.uint32(805459861).view(np.int32).item()


def _level_res():
    minres = np.array([16.0, 16.0, 16.0], dtype=np.float64)
    maxres = np.array([512.0, 512.0, 512.0], dtype=np.float64)
    b = np.exp((np.log(maxres) - np.log(minres)) / (NLEV - 1))
    return [int(np.floor(minres * b**l).astype(np.int64)[0]) for l in range(NLEV)]


RES = _level_res()

NW = 32
PPW = NPTS // NW
C = 256
NCHUNK = PPW // C
NG = C // 16
NGW = PPW // 16
TSLICE = TABLE_SIZE // 16

_i32 = jnp.int32
_f32 = jnp.float32


def _iota16():
    return lax.iota(_i32, 16)


def _round_half_even(u):
    t = u + 0.5
    r = t.astype(_i32)
    tie = (r.astype(_f32) == t) & ((r & 1) == 1)
    return jnp.where(tie, r - 1, r)


def _body(coords_hbm, grid_hbm, tables_hbm, res_hbm, out_hbm,
          cv, idx_a, idx_b, w_a, w_b, r0_a, r0_b, r1_a, r1_b,
          o0_a, o0_b, o1_a, o1_b, shared, res_v,
          sg0a, sg1a, sg0b, sg1b, sst0, sst1, sout):
    wid = lax.axis_index("s") * 2 + lax.axis_index("c")
    sid = lax.axis_index("s")
    pbase = wid * PPW

    pltpu.sync_copy(res_hbm, res_v)

    for d in range(3):
        pltpu.sync_copy(coords_hbm.at[pl.ds(d * NPTS + pbase, PPW)],
                        cv.at[pl.ds(d * PPW, PPW)])

    def norm(g, _):
        p = g * 16
        for d in range(3):
            q = d * PPW + p
            cv[pl.ds(q, 16)] = (cv[pl.ds(q, 16)] + 1.0) * 0.5
        return 0

    lax.fori_loop(0, NGW, norm, 0)

    def lvl0_chunk(ch, _):
        cb = ch * C
        r1f = float(RES[0] - 1)

        def g0(g, _):
            p = g * 16
            x = cv[pl.ds(0 * PPW + cb + p, 16)]
            y = cv[pl.ds(1 * PPW + cb + p, 16)]
            z = cv[pl.ds(2 * PPW + cb + p, 16)]
            ix = _round_half_even(x * r1f)
            iy = _round_half_even(y * r1f)
            iz = _round_half_even(z * r1f)
            gi = (ix * (RES[0] * RES[0]) + iy * RES[0] + iz) * 2
            idx_a[pl.ds(p, 16)] = gi
            idx_b[pl.ds(p, 16)] = gi + 1
            return 0

        lax.fori_loop(0, NG, g0, 0)
        d0 = pltpu.async_copy(grid_hbm.at[idx_a.at[pl.ds(0, C)]], o0_a, sg0a)
        d1 = pltpu.async_copy(grid_hbm.at[idx_b.at[pl.ds(0, C)]], o1_a, sg1a)
        d0.wait()
        d1.wait()
        pltpu.sync_copy(o0_a, out_hbm.at[pl.ds(0 * NPTS + pbase + cb, C)])
        pltpu.sync_copy(o1_a, out_hbm.at[pl.ds(1 * NPTS + pbase + cb, C)])
        return 0

    lax.fori_loop(0, NCHUNK, lvl0_chunk, 0)

    def index_gen(l, ch, idx_v, w_v):
        rm1 = res_v[pl.ds(l * 16, 16)] - 1
        rm1f = rm1.astype(_f32)
        cb = ch * C

        def ga(g, _):
            p = g * 16
            x = cv[pl.ds(0 * PPW + cb + p, 16)]
            y = cv[pl.ds(1 * PPW + cb + p, 16)]
            z = cv[pl.ds(2 * PPW + cb + p, 16)]
            ux = x * rm1f
            uy = y * rm1f
            uz = z * rm1f
            fx = ux.astype(_i32)
            fy = uy.astype(_i32)
            fz = uz.astype(_i32)
            wx = ux - fx.astype(_f32)
            wy = uy - fy.astype(_f32)
            wz = uz - fz.astype(_f32)
            x1 = jnp.minimum(fx + 1, rm1)
            y1 = jnp.minimum(fy + 1, rm1)
            z1 = jnp.minimum(fz + 1, rm1)
            hx = (fx, x1)
            hy = (fy * P1, y1 * P1)
            hz = (fz * P2, z1 * P2)
            ox = (1.0 - wx, wx)
            oy = (1.0 - wy, wy)
            oz = (1.0 - wz, wz)
            for ci, (dx, dy, dz) in enumerate(
                    [(a, b, c) for a in (0, 1) for b in (0, 1) for c in (0, 1)]):
                h = (hx[dx] ^ hy[dy] ^ hz[dz]) & TMASK
                wc = (ox[dx] * oy[dy]) * oz[dz]
                idx_v[pl.ds(ci * C + p, 16)] = h
                w_v[pl.ds(ci * C + p, 16)] = wc
            return 0

        lax.fori_loop(0, NG, ga, 0)

    def fire(idx_v, r0_v, r1_v, s0, s1):
        pltpu.async_copy(shared.at[pl.ds(0, TABLE_SIZE)].at[idx_v], r0_v, s0)
        pltpu.async_copy(
            shared.at[pl.ds(TABLE_SIZE, TABLE_SIZE)].at[idx_v], r1_v, s1)

    def drain(idx_v, r0_v, r1_v, s0, s1):
        pltpu.make_async_copy(
            shared.at[pl.ds(0, TABLE_SIZE)].at[idx_v], r0_v, s0).wait()
        pltpu.make_async_copy(
            shared.at[pl.ds(TABLE_SIZE, TABLE_SIZE)].at[idx_v], r1_v, s1).wait()

    def accumulate(l, ch, w_v, r0_v, r1_v, o0_v, o1_v):
        def gc(g, _):
            p = g * 16
            acc0 = jnp.zeros((16,), _f32)
            acc1 = jnp.zeros((16,), _f32)
            for ci in range(8):
                w = w_v[pl.ds(ci * C + p, 16)]
                acc0 = acc0 + w * r0_v[pl.ds(ci * C + p, 16)]
                acc1 = acc1 + w * r1_v[pl.ds(ci * C + p, 16)]
            o0_v[pl.ds(p, 16)] = acc0
            o1_v[pl.ds(p, 16)] = acc1
            return 0

        lax.fori_loop(0, NG, gc, 0)
        cb = ch * C
        do0 = pltpu.async_copy(
            o0_v, out_hbm.at[pl.ds((2 * l) * NPTS + pbase + cb, C)], sout)
        do1 = pltpu.async_copy(
            o1_v, out_hbm.at[pl.ds((2 * l + 1) * NPTS + pbase + cb, C)], sout)
        return do0, do1

    def level_body(l, _):
        plsc.subcore_barrier()
        toff = 2 * (l - 1) * TABLE_SIZE
        st0 = pltpu.async_copy(
            tables_hbm.at[pl.ds(toff + sid * TSLICE, TSLICE)],
            shared.at[pl.ds(sid * TSLICE, TSLICE)], sst0)
        st1 = pltpu.async_copy(
            tables_hbm.at[pl.ds(toff + TABLE_SIZE + sid * TSLICE, TSLICE)],
            shared.at[pl.ds(TABLE_SIZE + sid * TSLICE, TSLICE)], sst1)
        st0.wait()
        st1.wait()
        plsc.subcore_barrier()

        index_gen(l, 0, idx_a, w_a)
        fire(idx_a, r0_a, r1_a, sg0a, sg1a)

        def pair(i, _):
            chA = 2 * i
            index_gen(l, chA + 1, idx_b, w_b)
            drain(idx_a, r0_a, r1_a, sg0a, sg1a)
            fire(idx_b, r0_b, r1_b, sg0b, sg1b)
            oa = accumulate(l, chA, w_a, r0_a, r1_a, o0_a, o1_a)
            index_gen(l, chA + 2, idx_a, w_a)
            drain(idx_b, r0_b, r1_b, sg0b, sg1b)
            fire(idx_a, r0_a, r1_a, sg0a, sg1a)
            ob = accumulate(l, chA + 1, w_b, r0_b, r1_b, o0_b, o1_b)
            oa[0].wait()
            oa[1].wait()
            ob[0].wait()
            ob[1].wait()
            return 0

        lax.fori_loop(0, NCHUNK // 2 - 1, pair, 0)

        index_gen(l, NCHUNK - 1, idx_b, w_b)
        drain(idx_a, r0_a, r1_a, sg0a, sg1a)
        fire(idx_b, r0_b, r1_b, sg0b, sg1b)
        oa = accumulate(l, NCHUNK - 2, w_a, r0_a, r1_a, o0_a, o1_a)
        drain(idx_b, r0_b, r1_b, sg0b, sg1b)
        ob = accumulate(l, NCHUNK - 1, w_b, r0_b, r1_b, o0_b, o1_b)
        oa[0].wait()
        oa[1].wait()
        ob[0].wait()
        ob[1].wait()
        return 0

    lax.fori_loop(1, NLEV, level_body, 0)


@functools.partial(
    pl.kernel,
    out_type=jax.ShapeDtypeStruct((NOUT * NPTS,), _f32),
    mesh=plsc.VectorSubcoreMesh(core_axis_name="c", subcore_axis_name="s"),
    compiler_params=pltpu.CompilerParams(needs_layout_passes=False),
    scratch_types=[
        pltpu.VMEM((3 * PPW,), _f32),
        pltpu.VMEM((8 * C,), _i32),
        pltpu.VMEM((8 * C,), _i32),
        pltpu.VMEM((8 * C,), _f32),
        pltpu.VMEM((8 * C,), _f32),
        pltpu.VMEM((8 * C,), _f32),
        pltpu.VMEM((8 * C,), _f32),
        pltpu.VMEM((8 * C,), _f32),
        pltpu.VMEM((8 * C,), _f32),
        pltpu.VMEM((C,), _f32),
        pltpu.VMEM((C,), _f32),
        pltpu.VMEM((C,), _f32),
        pltpu.VMEM((C,), _f32),
        pltpu.VMEM_SHARED((2 * TABLE_SIZE,), _f32),
        pltpu.VMEM((NLEV * 16,), _i32),
        pltpu.SemaphoreType.DMA,
        pltpu.SemaphoreType.DMA,
        pltpu.SemaphoreType.DMA,
        pltpu.SemaphoreType.DMA,
        pltpu.SemaphoreType.DMA,
        pltpu.SemaphoreType.DMA,
        pltpu.SemaphoreType.DMA,
    ],
)
def _sc_encode(coords_hbm, grid_hbm, tables_hbm, res_hbm, out_hbm, *scratch):
    _body(coords_hbm, grid_hbm, tables_hbm, res_hbm, out_hbm, *scratch)


_RES_ARR = np.repeat(np.asarray(RES, dtype=np.int32), 16)


def kernel(coords, grid0, tables):
    coords_t = coords.T.reshape(-1)
    grid_r = grid0.transpose(1, 2, 3, 0).reshape(-1)
    tables_f = tables.reshape(-1)
    out = _sc_encode(coords_t, grid_r, tables_f, jnp.asarray(_RES_ARR))
    return out.reshape(NOUT, NPTS).T
